# baseline (speedup 1.0000x reference)
import functools

import jax
import jax.numpy as jnp
from jax.experimental import pallas as pl
from jax.experimental.pallas import tpu as pltpu

_MARGIN = 0.02
_EPS = 1e-8


def _gram_hinge_kernel(i_ref, j_ref, a_ref, b_ref, sum_ref, cnt_ref, *,
                       block, margin, eps):
    t = pl.program_id(0)

    @pl.when(t == 0)
    def _init():
        sum_ref[...] = jnp.zeros((1, 1), jnp.float32)
        cnt_ref[...] = jnp.zeros((1, 1), jnp.int32)

    a = a_ref[...]
    b = b_ref[...]
    inv_a = 1.0 / jnp.maximum(jnp.sqrt(jnp.sum(a * a, axis=1)), eps)
    inv_b = 1.0 / jnp.maximum(jnp.sqrt(jnp.sum(b * b, axis=1)), eps)
    sim = jax.lax.dot_general(
        a, b, (((1,), (1,)), ((), ())),
        preferred_element_type=jnp.float32,
        precision=jax.lax.Precision.HIGHEST,
    )
    sim = sim * inv_a[:, None] * inv_b[None, :]

    i = i_ref[t]
    j = j_ref[t]
    rows = i * block + jax.lax.broadcasted_iota(jnp.int32, sim.shape, 0)
    cols = j * block + jax.lax.broadcasted_iota(jnp.int32, sim.shape, 1)
    pos = jnp.logical_and(sim > margin, cols > rows)
    sum_ref[...] += jnp.sum(jnp.where(pos, sim - margin, 0.0)).reshape(1, 1)
    cnt_ref[...] += jnp.sum(pos.astype(jnp.int32)).reshape(1, 1)


def _layer_hinge_sums(w, block):
    n, d = w.shape
    assert n % block == 0
    nb = n // block
    pairs = [(i, j) for i in range(nb) for j in range(nb) if j >= i]
    num_steps = len(pairs)
    i_idx = jnp.asarray([p[0] for p in pairs], jnp.int32)
    j_idx = jnp.asarray([p[1] for p in pairs], jnp.int32)

    grid_spec = pltpu.PrefetchScalarGridSpec(
        num_scalar_prefetch=2,
        grid=(num_steps,),
        in_specs=[
            pl.BlockSpec((block, d), lambda t, ii, jj: (ii[t], 0)),
            pl.BlockSpec((block, d), lambda t, ii, jj: (jj[t], 0)),
        ],
        out_specs=[
            pl.BlockSpec((1, 1), lambda t, ii, jj: (0, 0)),
            pl.BlockSpec((1, 1), lambda t, ii, jj: (0, 0)),
        ],
    )
    s, c = pl.pallas_call(
        functools.partial(_gram_hinge_kernel, block=block, margin=_MARGIN,
                          eps=_EPS),
        grid_spec=grid_spec,
        out_shape=[
            jax.ShapeDtypeStruct((1, 1), jnp.float32),
            jax.ShapeDtypeStruct((1, 1), jnp.int32),
        ],
    )(i_idx, j_idx, w, w)
    return s[0, 0], c[0, 0]


def kernel(w0, w1):
    s0, c0 = _layer_hinge_sums(w0, 512)
    s1, c1 = _layer_hinge_sums(w1, 512)
    l0 = s0 / jnp.maximum(c0, 1).astype(jnp.float32)
    l1 = s1 / jnp.maximum(c1, 1).astype(jnp.float32)
    return 0.5 * (l0 + l1)


# speedup vs baseline: 2.7762x; 2.7762x over previous
import functools

import jax
import jax.numpy as jnp
from jax.experimental import pallas as pl
from jax.experimental.pallas import tpu as pltpu

_MARGIN = 0.02
_EPS = 1e-8


def _gram_hinge_kernel(i_ref, j_ref, a_ref, b_ref, sum_ref, cnt_ref, *,
                       block, margin, eps):
    t = pl.program_id(0)

    @pl.when(t == 0)
    def _init():
        sum_ref[...] = jnp.zeros((1, 1), jnp.float32)
        cnt_ref[...] = jnp.zeros((1, 1), jnp.int32)

    a = a_ref[...]
    b = b_ref[...]
    inv_a = 1.0 / jnp.maximum(jnp.sqrt(jnp.sum(a * a, axis=1)), eps)
    inv_b = 1.0 / jnp.maximum(jnp.sqrt(jnp.sum(b * b, axis=1)), eps)
    sim = jax.lax.dot_general(
        a, b, (((1,), (1,)), ((), ())),
        preferred_element_type=jnp.float32,
    )
    sim = sim * inv_a[:, None] * inv_b[None, :]

    i = i_ref[t]
    j = j_ref[t]
    rows = i * block + jax.lax.broadcasted_iota(jnp.int32, sim.shape, 0)
    cols = j * block + jax.lax.broadcasted_iota(jnp.int32, sim.shape, 1)
    pos = jnp.logical_and(sim > margin, cols > rows)
    sum_ref[...] += jnp.sum(jnp.where(pos, sim - margin, 0.0)).reshape(1, 1)
    cnt_ref[...] += jnp.sum(pos.astype(jnp.int32)).reshape(1, 1)


def _layer_hinge_sums(w, block):
    n, d = w.shape
    assert n % block == 0
    nb = n // block
    pairs = [(i, j) for i in range(nb) for j in range(nb) if j >= i]
    num_steps = len(pairs)
    i_idx = jnp.asarray([p[0] for p in pairs], jnp.int32)
    j_idx = jnp.asarray([p[1] for p in pairs], jnp.int32)

    grid_spec = pltpu.PrefetchScalarGridSpec(
        num_scalar_prefetch=2,
        grid=(num_steps,),
        in_specs=[
            pl.BlockSpec((block, d), lambda t, ii, jj: (ii[t], 0)),
            pl.BlockSpec((block, d), lambda t, ii, jj: (jj[t], 0)),
        ],
        out_specs=[
            pl.BlockSpec((1, 1), lambda t, ii, jj: (0, 0)),
            pl.BlockSpec((1, 1), lambda t, ii, jj: (0, 0)),
        ],
    )
    s, c = pl.pallas_call(
        functools.partial(_gram_hinge_kernel, block=block, margin=_MARGIN,
                          eps=_EPS),
        grid_spec=grid_spec,
        out_shape=[
            jax.ShapeDtypeStruct((1, 1), jnp.float32),
            jax.ShapeDtypeStruct((1, 1), jnp.int32),
        ],
    )(i_idx, j_idx, w, w)
    return s[0, 0], c[0, 0]


def kernel(w0, w1):
    s0, c0 = _layer_hinge_sums(w0, 512)
    s1, c1 = _layer_hinge_sums(w1, 512)
    l0 = s0 / jnp.maximum(c0, 1).astype(jnp.float32)
    l1 = s1 / jnp.maximum(c1, 1).astype(jnp.float32)
    return 0.5 * (l0 + l1)
